# baseline (speedup 1.0000x reference)
import functools

import jax
import jax.numpy as jnp
from jax import lax
from jax.experimental import pallas as pl
from jax.experimental.pallas import tpu as pltpu
from jax.experimental.pallas import tpu_sc as plsc

N_NODES = 10000
N_EDGES = 320000
F_IN = 128
OUT_SCALE = 1.0 / 128.0

NW = 32
LANES = 128
PER_TILE = 10240
E_PAD = NW * PER_TILE
CHUNK = 512
SUB = CHUNK // LANES
N_CHUNKS = PER_TILE // CHUNK
TABLE_N = 10240
ZERO_PER_TILE = TABLE_N // 16


def _sc_edge_body(y_hbm, ed_hbm, out_hbm,
                  edb, eidx, cref, rows, msg, table, semg, seme, sems):
    cid = lax.axis_index("c")
    sid = lax.axis_index("s")
    wid = (1 - cid) * 16 + sid

    def zero_body(i, _):
        msg[0, i] = jnp.zeros((16,), jnp.float32)
        msg[1, i] = jnp.zeros((16,), jnp.float32)
        return 0
    lax.fori_loop(0, CHUNK, zero_body, 0)
    pltpu.sync_copy(msg.at[0], table.at[pl.ds(sid * ZERO_PER_TILE, CHUNK)])
    pltpu.sync_copy(msg.at[1, pl.ds(0, ZERO_PER_TILE - CHUNK)],
                    table.at[pl.ds(sid * ZERO_PER_TILE + CHUNK,
                                   ZERO_PER_TILE - CHUNK)])
    plsc.subcore_barrier()

    base = wid * (PER_TILE // LANES)
    lane = jnp.arange(16, dtype=jnp.int32)
    zero16 = jnp.zeros((16,), jnp.int32)
    one16 = jnp.ones((16,), jnp.int32)

    def load_edata(g):
        return pltpu.async_copy(
            ed_hbm.at[pl.ds(base + g * SUB, SUB)], edb.at[g % 2], seme)

    def fire_gathers(g):
        eb = edb.at[g % 2]
        rb = rows.at[g % 2]
        return [pltpu.async_copy(y_hbm.at[eb.at[j, 1]],
                                 rb.at[pl.ds(j * LANES, LANES)], semg)
                for j in range(SUB)]

    def coef_compute(g):
        eb = edb.at[g % 2]
        ex = eidx.at[g % 2]

        def coef_body(k, _):
            r = lax.shift_right_logical(k, 3)
            col = pl.ds((k & 7) * 16, 16)
            ei_v = eb[r, 0, col]
            ej_v = eb[r, 1, col]
            a = jnp.clip(plsc.bitcast(eb[r, 2, col], jnp.float32),
                         -1.0, 1.0)
            b = jnp.clip(plsc.bitcast(eb[r, 3, col], jnp.float32),
                         -1.0, 1.0)
            erow = k * 16 + lane
            plsc.store_scatter(cref, [erow, zero16], (a + 1.0) * 0.5)
            plsc.store_scatter(cref, [erow, one16], (b + 1.0) * 0.5)
            ex[r, col] = jnp.where(ei_v != ej_v, ei_v, N_NODES)
            return 0
        lax.fori_loop(0, CHUNK // 16, coef_body, 0)

    def edge_compute(g):
        mb = msg.at[g % 2]
        rb = rows.at[g % 2]

        @plsc.parallel_loop(0, CHUNK, 1, unroll=8)
        def edge_body(e):
            c = cref[e]
            tu = c[0]
            tv = c[1]
            s0 = rb[e, pl.ds(0, 16)]
            s1 = rb[e, pl.ds(16, 16)]
            s2 = rb[e, pl.ds(32, 16)]
            s3 = rb[e, pl.ds(48, 16)]
            h0 = s0 + tu * (s2 - s0)
            h1 = s1 + tu * (s3 - s1)
            mb[e] = h0 + tv * (h1 - h0)

    def fire_scatters(g):
        mb = msg.at[g % 2]
        ex = eidx.at[g % 2]
        return [pltpu.async_copy(mb.at[pl.ds(j * LANES, LANES)],
                                 table.at[ex.at[j]], sems, add=True)
                for j in range(SUB)]

    load_edata(0).wait()
    gathers = fire_gathers(0)
    ed_next = load_edata(1)
    scatters = {}
    for g in range(N_CHUNKS):
        with jax.named_scope("wait_gather"):
            for cp in gathers:
                cp.wait()
        with jax.named_scope("coef"):
            coef_compute(g)
        with jax.named_scope("fire_next"):
            if g + 1 < N_CHUNKS:
                ed_next.wait()
                gathers = fire_gathers(g + 1)
            if g + 2 < N_CHUNKS:
                ed_next = load_edata(g + 2)
        with jax.named_scope("wait_scat"):
            for cp in scatters.pop(g % 2, ()):
                cp.wait()
        with jax.named_scope("edge"):
            edge_compute(g)
        with jax.named_scope("fire_scat"):
            scatters[g % 2] = fire_scatters(g)
    for cps in scatters.values():
        for cp in cps:
            cp.wait()

    plsc.subcore_barrier()

    pltpu.sync_copy(table.at[pl.ds(sid * ZERO_PER_TILE, ZERO_PER_TILE)],
                    out_hbm.at[cid, pl.ds(sid * ZERO_PER_TILE, ZERO_PER_TILE)])


_sc_edge = functools.partial(
    pl.kernel,
    out_type=jax.ShapeDtypeStruct((2, TABLE_N, 16), jnp.float32),
    mesh=plsc.VectorSubcoreMesh(core_axis_name="c", subcore_axis_name="s"),
    compiler_params=pltpu.CompilerParams(needs_layout_passes=False,
                                         use_tc_tiling_on_sc=False),
    scratch_types=[
        pltpu.VMEM((2, SUB, 4, LANES), jnp.int32),
        pltpu.VMEM((2, SUB, LANES), jnp.int32),
        pltpu.VMEM((CHUNK, 16), jnp.float32),
        pltpu.VMEM((2, CHUNK, 64), jnp.float32),
        pltpu.VMEM((2, CHUNK, 16), jnp.float32),
        pltpu.VMEM_SHARED((TABLE_N, 16), jnp.float32),
        pltpu.SemaphoreType.DMA,
        pltpu.SemaphoreType.DMA,
        pltpu.SemaphoreType.DMA,
    ],
)(_sc_edge_body)


_GRID = 10
_BLK = N_NODES // _GRID


def _tc0_body(x_ref, w_ref, b_ref, o_ref):
    o_ref[...] = (jnp.dot(x_ref[...], w_ref[...],
                          preferred_element_type=jnp.float32)
                  + b_ref[0:1, :])


def _tc_mid_body(prev_ref, part_ref, w_ref, b_ref, o_ref):
    lin = jax.nn.relu(prev_ref[:, 64:80])
    conv = jax.nn.relu(part_ref[0] + part_ref[1])
    o_ref[...] = (jnp.dot(lin, w_ref[0:16, :],
                          preferred_element_type=jnp.float32)
                  + jnp.dot(conv, w_ref[16:32, :],
                            preferred_element_type=jnp.float32)
                  + b_ref[0:1, :])


def _tc2_body(prev_ref, part_ref, w_ref, b_ref, o_ref):
    ans = jax.nn.relu(prev_ref[:, 64:80] + part_ref[0] + part_ref[1])
    o_ref[...] = (jnp.dot(ans, w_ref[...],
                          preferred_element_type=jnp.float32)
                  + b_ref[0:1, :])


def _tc3_body(prev_ref, part_ref, o_ref):
    o_ref[...] = (prev_ref[:, 64:66]
                  + part_ref[0, :, 0:2] + part_ref[1, :, 0:2]) * OUT_SCALE


def _mm(body, n_in, kdims, out_w):
    in_specs = []
    for kd in kdims:
        if kd == "rows":
            in_specs.append(pl.BlockSpec((_BLK, n_in), lambda i: (i, 0)))
        elif kd == "part":
            in_specs.append(pl.BlockSpec((2, _BLK, 16), lambda i: (0, i, 0)))
        elif isinstance(kd, tuple):
            in_specs.append(pl.BlockSpec(kd, lambda i: (0, 0)))
    return pl.pallas_call(
        body,
        grid=(_GRID,),
        in_specs=in_specs,
        out_specs=pl.BlockSpec((_BLK, out_w), lambda i: (i, 0)),
        out_shape=jax.ShapeDtypeStruct((N_NODES, out_w), jnp.float32),
    )


def kernel(x, edge_index, edge_attr, Wc0, Wf0, bf0, Wc1, Wf1, bf1,
           Wc2, Wf2, bf2):
    f32 = jnp.float32

    def stack_conv(Wc, out_pad):
        blocks = [Wc[u, v] for u in range(2) for v in range(2)]
        if out_pad:
            blocks = [jnp.pad(b, ((0, 0), (0, out_pad))) for b in blocks]
        return jnp.concatenate(blocks, axis=1)

    w0 = jnp.concatenate([stack_conv(Wc0, 0), Wf0], axis=1)
    b0 = jnp.concatenate([jnp.zeros((64,), f32), bf0])
    w1 = jnp.concatenate([stack_conv(Wc1, 0), Wf1], axis=1)
    b1 = jnp.concatenate([jnp.zeros((64,), f32), bf1])
    w2 = jnp.concatenate([stack_conv(Wc2, 14),
                          jnp.pad(Wf2, ((0, 0), (0, 14)))], axis=1)
    b2 = jnp.concatenate([jnp.zeros((64,), f32), bf2,
                          jnp.zeros((14,), f32)])
    b0 = jnp.broadcast_to(b0, (8, 80))
    b1 = jnp.broadcast_to(b1, (8, 80))
    b2 = jnp.broadcast_to(b2, (8, 80))

    pad = E_PAD - N_EDGES

    def lanes128(v):
        return jnp.pad(v, (0, pad)).reshape(E_PAD // LANES, 1, LANES)

    ed = jnp.concatenate([
        lanes128(edge_index[0]),
        lanes128(edge_index[1]),
        lanes128(lax.bitcast_convert_type(edge_attr[:, 0], jnp.int32)),
        lanes128(lax.bitcast_convert_type(edge_attr[:, 1], jnp.int32)),
    ], axis=1)

    out0 = _mm(_tc0_body, F_IN, ["rows", (F_IN, 80), (8, 80)], 80)(
        x, w0, b0)
    part0 = _sc_edge(out0[:, 0:64], ed)[:, :N_NODES]

    out1 = _mm(_tc_mid_body, 80, ["rows", "part", (32, 80), (8, 80)], 80)(
        out0, part0, w1, b1)
    part1 = _sc_edge(out1[:, 0:64], ed)[:, :N_NODES]

    out2 = _mm(_tc2_body, 80, ["rows", "part", (16, 80), (8, 80)], 80)(
        out1, part1, w2, b2)
    part2 = _sc_edge(out2[:, 0:64], ed)[:, :N_NODES]

    return _mm(_tc3_body, 80, ["rows", "part"], 2)(out2, part2)


# speedup vs baseline: 2.6210x; 2.6210x over previous
import functools

import jax
import jax.numpy as jnp
from jax import lax
from jax.experimental import pallas as pl
from jax.experimental.pallas import tpu as pltpu
from jax.experimental.pallas import tpu_sc as plsc

N_NODES = 10000
N_EDGES = 320000
F_IN = 128
OUT_SCALE = 1.0 / 128.0

NW = 32
LANES = 128
PER_TILE = 10240
E_PAD = NW * PER_TILE
CHUNK = 512
SUB = CHUNK // LANES
N_CHUNKS = PER_TILE // CHUNK
TABLE_N = 10240
ZERO_PER_TILE = TABLE_N // 16


def _sc_edge_body(y_hbm, ed_hbm, out_hbm,
                  edb, eidx, cref, rows, msg, table, semg, seme, sems):
    cid = lax.axis_index("c")
    sid = lax.axis_index("s")
    wid = (1 - cid) * 16 + sid

    def zero_body(i, _):
        msg[0, i] = jnp.zeros((16,), jnp.float32)
        msg[1, i] = jnp.zeros((16,), jnp.float32)
        return 0
    lax.fori_loop(0, CHUNK, zero_body, 0)
    pltpu.sync_copy(msg.at[0], table.at[pl.ds(sid * ZERO_PER_TILE, CHUNK)])
    pltpu.sync_copy(msg.at[1, pl.ds(0, ZERO_PER_TILE - CHUNK)],
                    table.at[pl.ds(sid * ZERO_PER_TILE + CHUNK,
                                   ZERO_PER_TILE - CHUNK)])
    plsc.subcore_barrier()

    base = wid * (PER_TILE // LANES)
    lane = jnp.arange(16, dtype=jnp.int32)
    zero16 = jnp.zeros((16,), jnp.int32)
    one16 = jnp.ones((16,), jnp.int32)

    def load_edata(g):
        return pltpu.async_copy(
            ed_hbm.at[pl.ds(base + g * SUB, SUB)], edb.at[g % 2], seme)

    def fire_gathers(g):
        eb = edb.at[g % 2]
        rb = rows.at[g % 2]
        return [pltpu.async_copy(y_hbm.at[eb.at[j, 1]],
                                 rb.at[pl.ds(j * LANES, LANES)], semg)
                for j in range(SUB)]

    def coef_compute(g):
        eb = edb.at[g % 2]
        ex = eidx.at[g % 2]

        def coef_body(k, _):
            r = lax.shift_right_logical(k, 3)
            col = pl.ds((k & 7) * 16, 16)
            ei_v = eb[r, 0, col]
            ej_v = eb[r, 1, col]
            a = jnp.clip(plsc.bitcast(eb[r, 2, col], jnp.float32),
                         -1.0, 1.0)
            b = jnp.clip(plsc.bitcast(eb[r, 3, col], jnp.float32),
                         -1.0, 1.0)
            erow = k * 16 + lane
            plsc.store_scatter(cref, [erow, zero16], (a + 1.0) * 0.5)
            plsc.store_scatter(cref, [erow, one16], (b + 1.0) * 0.5)
            ex[r, col] = jnp.where(ei_v != ej_v, ei_v, N_NODES)
            return 0
        lax.fori_loop(0, CHUNK // 16, coef_body, 0)

    def edge_compute(g):
        mb = msg.at[g % 2]
        rb = rows.at[g % 2]

        @plsc.parallel_loop(0, CHUNK, 1, unroll=8)
        def edge_body(e):
            c = cref[e]
            tu = c[0]
            tv = c[1]
            s0 = rb[e, pl.ds(0, 16)]
            s1 = rb[e, pl.ds(16, 16)]
            s2 = rb[e, pl.ds(32, 16)]
            s3 = rb[e, pl.ds(48, 16)]
            h0 = s0 + tu * (s2 - s0)
            h1 = s1 + tu * (s3 - s1)
            mb[e] = h0 + tv * (h1 - h0)

    def fire_scatters(g):
        mb = msg.at[g % 2]
        ex = eidx.at[g % 2]
        return [pltpu.async_copy(mb.at[pl.ds(j * LANES, LANES)],
                                 table.at[ex.at[j]], sems, add=True)
                for j in range(SUB)]

    load_edata(0).wait()
    gathers = fire_gathers(0)
    ed_next = load_edata(1)
    scatters = {}
    for g in range(N_CHUNKS):
        with jax.named_scope("wait_gather"):
            for cp in gathers:
                cp.wait()
        with jax.named_scope("coef"):
            coef_compute(g)
        with jax.named_scope("fire_next"):
            if g + 1 < N_CHUNKS:
                ed_next.wait()
                gathers = fire_gathers(g + 1)
            if g + 2 < N_CHUNKS:
                ed_next = load_edata(g + 2)
        with jax.named_scope("wait_scat"):
            for cp in scatters.pop(g % 2, ()):
                cp.wait()
        with jax.named_scope("edge"):
            edge_compute(g)
        with jax.named_scope("fire_scat"):
            scatters[g % 2] = fire_scatters(g)
    for cps in scatters.values():
        for cp in cps:
            cp.wait()

    plsc.subcore_barrier()

    pltpu.sync_copy(table.at[pl.ds(sid * ZERO_PER_TILE, ZERO_PER_TILE)],
                    out_hbm.at[cid, pl.ds(sid * ZERO_PER_TILE, ZERO_PER_TILE)])


_sc_edge = functools.partial(
    pl.kernel,
    out_type=jax.ShapeDtypeStruct((2, TABLE_N, 16), jnp.float32),
    mesh=plsc.VectorSubcoreMesh(core_axis_name="c", subcore_axis_name="s"),
    compiler_params=pltpu.CompilerParams(needs_layout_passes=False,
                                         use_tc_tiling_on_sc=False),
    scratch_types=[
        pltpu.VMEM((2, SUB, 4, LANES), jnp.int32),
        pltpu.VMEM((2, SUB, LANES), jnp.int32),
        pltpu.VMEM((CHUNK, 16), jnp.float32),
        pltpu.VMEM((2, CHUNK, 64), jnp.float32),
        pltpu.VMEM((2, CHUNK, 16), jnp.float32),
        pltpu.VMEM_SHARED((TABLE_N, 16), jnp.float32),
        pltpu.SemaphoreType.DMA,
        pltpu.SemaphoreType.DMA,
        pltpu.SemaphoreType.DMA,
    ],
)(_sc_edge_body)


_GRID = 10
_BLK = N_NODES // _GRID


def _tc0_body(x_ref, w_ref, b_ref, o_ref):
    o_ref[...] = (jnp.dot(x_ref[...], w_ref[...],
                          preferred_element_type=jnp.float32)
                  + b_ref[0:1, :])


def _tc_mid_body(prev_ref, part_ref, w_ref, b_ref, o_ref):
    lin = jax.nn.relu(prev_ref[:, 64:80])
    conv = jax.nn.relu(part_ref[0] + part_ref[1])
    o_ref[...] = (jnp.dot(lin, w_ref[0:16, :],
                          preferred_element_type=jnp.float32)
                  + jnp.dot(conv, w_ref[16:32, :],
                            preferred_element_type=jnp.float32)
                  + b_ref[0:1, :])


def _tc2_body(prev_ref, part_ref, w_ref, b_ref, o_ref):
    ans = jax.nn.relu(prev_ref[:, 64:80] + part_ref[0] + part_ref[1])
    o_ref[...] = (jnp.dot(ans, w_ref[...],
                          preferred_element_type=jnp.float32)
                  + b_ref[0:1, :])


def _tc3_body(prev_ref, part_ref, o_ref):
    o_ref[...] = (prev_ref[:, 64:66]
                  + part_ref[0, :, 0:2] + part_ref[1, :, 0:2]) * OUT_SCALE


def _mm(body, n_in, kdims, out_w):
    in_specs = []
    for kd in kdims:
        if kd == "rows":
            in_specs.append(pl.BlockSpec((_BLK, n_in), lambda i: (i, 0)))
        elif kd == "part":
            in_specs.append(pl.BlockSpec((2, _BLK, 16), lambda i: (0, i, 0)))
        elif isinstance(kd, tuple):
            in_specs.append(pl.BlockSpec(kd, lambda i: (0, 0)))
    return pl.pallas_call(
        body,
        grid=(_GRID,),
        in_specs=in_specs,
        out_specs=pl.BlockSpec((_BLK, out_w), lambda i: (i, 0)),
        out_shape=jax.ShapeDtypeStruct((N_NODES, out_w), jnp.float32),
    )


def kernel(x, edge_index, edge_attr, Wc0, Wf0, bf0, Wc1, Wf1, bf1,
           Wc2, Wf2, bf2):
    f32 = jnp.float32

    def stack_conv(Wc, out_pad):
        blocks = [Wc[u, v] for u in range(2) for v in range(2)]
        if out_pad:
            blocks = [jnp.pad(b, ((0, 0), (0, out_pad))) for b in blocks]
        return jnp.concatenate(blocks, axis=1)

    w0 = jnp.concatenate([stack_conv(Wc0, 0), Wf0], axis=1)
    b0 = jnp.concatenate([jnp.zeros((64,), f32), bf0])
    w1 = jnp.concatenate([stack_conv(Wc1, 0), Wf1], axis=1)
    b1 = jnp.concatenate([jnp.zeros((64,), f32), bf1])
    w2 = jnp.concatenate([stack_conv(Wc2, 14),
                          jnp.pad(Wf2, ((0, 0), (0, 14)))], axis=1)
    b2 = jnp.concatenate([jnp.zeros((64,), f32), bf2,
                          jnp.zeros((14,), f32)])
    b0 = jnp.broadcast_to(b0, (8, 80))
    b1 = jnp.broadcast_to(b1, (8, 80))
    b2 = jnp.broadcast_to(b2, (8, 80))

    pad = E_PAD - N_EDGES
    pad_idx = jnp.arange(pad, dtype=jnp.int32) % N_NODES

    def lanes128(v, fill):
        return jnp.concatenate([v, fill]).reshape(E_PAD // LANES, 1, LANES)

    zpad = jnp.zeros((pad,), jnp.int32)
    ed = jnp.concatenate([
        lanes128(edge_index[0], pad_idx),
        lanes128(edge_index[1], pad_idx),
        lanes128(lax.bitcast_convert_type(edge_attr[:, 0], jnp.int32), zpad),
        lanes128(lax.bitcast_convert_type(edge_attr[:, 1], jnp.int32), zpad),
    ], axis=1)

    out0 = _mm(_tc0_body, F_IN, ["rows", (F_IN, 80), (8, 80)], 80)(
        x, w0, b0)
    part0 = _sc_edge(out0[:, 0:64], ed)[:, :N_NODES]

    out1 = _mm(_tc_mid_body, 80, ["rows", "part", (32, 80), (8, 80)], 80)(
        out0, part0, w1, b1)
    part1 = _sc_edge(out1[:, 0:64], ed)[:, :N_NODES]

    out2 = _mm(_tc2_body, 80, ["rows", "part", (16, 80), (8, 80)], 80)(
        out1, part1, w2, b2)
    part2 = _sc_edge(out2[:, 0:64], ed)[:, :N_NODES]

    return _mm(_tc3_body, 80, ["rows", "part"], 2)(out2, part2)
